# baseline (speedup 1.0000x reference)
import jax
import jax.numpy as jnp
from jax.experimental import pallas as pl

E = 64
TOP_K = 8
N_GROUP = 8
TOPK_GROUP = 3
GROUP_SIZE = E // N_GROUP
SCALE = 16.0

BT = 1024


def _gate_kernel(x_ref, w_ref, idx_ref, wgt_ref):
    x = x_ref[...]
    w = w_ref[...]
    logits = jax.lax.dot_general(
        x, w, (((1,), (1,)), ((), ())),
        preferred_element_type=jnp.float32,
    )

    lt = logits.T
    m = jnp.max(lt, axis=0, keepdims=True)
    ex = jnp.exp(lt - m)
    scores = ex / jnp.sum(ex, axis=0, keepdims=True)

    bt = scores.shape[1]
    gsf = jnp.max(scores.reshape(N_GROUP, GROUP_SIZE, bt), axis=1)

    giota = jax.lax.broadcasted_iota(jnp.int32, (N_GROUP, bt), 0)
    gmask = jnp.zeros((N_GROUP, bt), jnp.bool_)
    gwork = gsf
    for _ in range(TOPK_GROUP):
        gm = jnp.max(gwork, axis=0, keepdims=True)
        eq = gwork == gm
        first = jnp.min(jnp.where(eq, giota, N_GROUP), axis=0, keepdims=True)
        sel = giota == first
        gmask = jnp.logical_or(gmask, sel)
        gwork = jnp.where(sel, -1.0, gwork)

    emask = jnp.broadcast_to(
        gmask.reshape(N_GROUP, 1, bt), (N_GROUP, GROUP_SIZE, bt)
    ).reshape(E, bt)
    tmp = jnp.where(emask, scores, -1.0)

    eiota = jax.lax.broadcasted_iota(jnp.int32, (E, bt), 0)
    idx_rows, wgt_rows = [], []
    for _ in range(TOP_K):
        km = jnp.max(tmp, axis=0, keepdims=True)
        eq = tmp == km
        first = jnp.min(jnp.where(eq, eiota, E), axis=0, keepdims=True)
        idx_rows.append(first)
        wgt_rows.append(km)
        tmp = jnp.where(eiota == first, -1.0, tmp)

    idx_ref[...] = jnp.concatenate(idx_rows, axis=0)
    wgt_ref[...] = jnp.concatenate(wgt_rows, axis=0) * SCALE


def kernel(hidden_states, weight):
    bsz, seq_len, hidden_dim = hidden_states.shape
    n_tokens = bsz * seq_len
    x = hidden_states.reshape(n_tokens, hidden_dim).astype(jnp.float32)
    w = weight.astype(jnp.float32)

    grid = (n_tokens // BT,)
    idx_t, wgt_t = pl.pallas_call(
        _gate_kernel,
        grid=grid,
        in_specs=[
            pl.BlockSpec((BT, hidden_dim), lambda i: (i, 0)),
            pl.BlockSpec((E, hidden_dim), lambda i: (0, 0)),
        ],
        out_specs=[
            pl.BlockSpec((TOP_K, BT), lambda i: (0, i)),
            pl.BlockSpec((TOP_K, BT), lambda i: (0, i)),
        ],
        out_shape=[
            jax.ShapeDtypeStruct((TOP_K, n_tokens), jnp.int32),
            jax.ShapeDtypeStruct((TOP_K, n_tokens), jnp.float32),
        ],
    )(x, w)
    return idx_t.T, wgt_t.T
